# Initial kernel scaffold; baseline (speedup 1.0000x reference)
import jax
import jax.numpy as jnp
from jax.experimental import pallas as pl


def kernel(x, edge_index, edge_attr, W0, b0, We0, be0, W1_0, b1_0, W2_0, b2_0, gamma0, beta0, We1, be1, W1_1, b1_1, W2_1, b2_1, gamma1, beta1):
    raise NotImplementedError("write your pallas kernel here")


# speedup vs baseline: 4.9120x; 4.9120x over previous
import functools

import jax
import jax.numpy as jnp
from jax import lax
from jax.experimental import pallas as pl
from jax.experimental.pallas import tpu as pltpu
from jax.experimental.pallas import tpu_sc as plsc

N = 10000
E = 320000
D = 128
DE = 16
H = 256

NC = 2
NS = 16
NW = NC * NS
EPW = E // NW
CH = 128
NCHUNK = (EPW + CH - 1) // CH
EPW_PAD = NCHUNK * CH
N_PAD = 10112
RPT = N_PAD // NS

_mesh = plsc.VectorSubcoreMesh(
    core_axis_name="c", subcore_axis_name="s", num_cores=NC, num_subcores=NS)

_f32 = jnp.float32


def _zero16():
    return jnp.zeros((16,), _f32)


def _sc_edge_aggr_body(dst3, ea4, ed_out, dstv, rows, zbuf, acc, sem):
    c = lax.axis_index("c")
    s = lax.axis_index("s")
    w = c * NS + s

    one0 = jnp.where(lax.iota(jnp.int32, 16) == 0, 1.0, 0.0).astype(_f32)

    def _init_rows(i, _):
        rows[i, pl.ds(0, 16)] = _zero16()
        rows[i, pl.ds(16, 16)] = one0
        return 0

    lax.fori_loop(0, CH, _init_rows, 0)

    def _init_z(i, _):
        zbuf[i, pl.ds(0, 16)] = _zero16()
        zbuf[i, pl.ds(16, 16)] = _zero16()
        return 0

    lax.fori_loop(0, RPT, _init_z, 0)

    pltpu.sync_copy(zbuf, acc.at[pl.ds(s * RPT, RPT)])
    plsc.subcore_barrier()

    pltpu.sync_copy(dst3.at[w], dstv)

    def _step(k, _):
        pltpu.sync_copy(ea4.at[w, k], rows.at[:, pl.ds(0, DE)])
        pltpu.sync_copy(rows, acc.at[dstv.at[k]], add=True)
        return 0

    lax.fori_loop(0, NCHUNK, _step, 0)
    plsc.subcore_barrier()

    pltpu.sync_copy(acc.at[pl.ds(s * RPT, RPT)],
                    ed_out.at[c, pl.ds(s * RPT, RPT)])


_sc_edge_aggr = functools.partial(
    pl.kernel,
    out_type=jax.ShapeDtypeStruct((NC, N_PAD, 2 * DE), _f32),
    mesh=_mesh,
    scratch_types=[
        pltpu.VMEM((NCHUNK, CH), jnp.int32),
        pltpu.VMEM((CH, 2 * DE), _f32),
        pltpu.VMEM((RPT, 2 * DE), _f32),
        pltpu.VMEM_SHARED((N_PAD, 2 * DE), _f32),
        pltpu.SemaphoreType.DMA,
    ],
    compiler_params=pltpu.CompilerParams(use_tc_tiling_on_sc=False),
)(_sc_edge_aggr_body)


def _sc_gather_scatter_body(h_hbm, src3, dst3, s_out, srcv, dstv, rows, acc, sem):
    c = lax.axis_index("c")
    s = lax.axis_index("s")
    w = c * NS + s

    def _zero_rows(i, _):
        for j in range(D // 16):
            rows[i, pl.ds(j * 16, 16)] = _zero16()
        return 0

    lax.fori_loop(0, CH, _zero_rows, 0)

    base = s * RPT
    for off, sz in ((0, CH), (CH, CH), (2 * CH, CH), (3 * CH, CH), (4 * CH, RPT - 4 * CH)):
        pltpu.sync_copy(rows.at[pl.ds(0, sz)], acc.at[pl.ds(base + off, sz)])
    plsc.subcore_barrier()

    pltpu.sync_copy(src3.at[w], srcv)
    pltpu.sync_copy(dst3.at[w], dstv)

    def _step(k, _):
        pltpu.async_copy(h_hbm.at[srcv.at[k]], rows, sem).wait()
        pltpu.sync_copy(rows, acc.at[dstv.at[k]], add=True)
        return 0

    lax.fori_loop(0, NCHUNK, _step, 0)
    plsc.subcore_barrier()

    for off, sz in ((0, CH), (CH, CH), (2 * CH, CH), (3 * CH, CH), (4 * CH, RPT - 4 * CH)):
        pltpu.sync_copy(acc.at[pl.ds(base + off, sz)],
                        s_out.at[c, pl.ds(base + off, sz)])


_sc_gather_scatter = functools.partial(
    pl.kernel,
    out_type=jax.ShapeDtypeStruct((NC, N_PAD, D), _f32),
    mesh=_mesh,
    scratch_types=[
        pltpu.VMEM((NCHUNK, CH), jnp.int32),
        pltpu.VMEM((NCHUNK, CH), jnp.int32),
        pltpu.VMEM((CH, D), _f32),
        pltpu.VMEM_SHARED((N_PAD, D), _f32),
        pltpu.SemaphoreType.DMA,
    ],
    compiler_params=pltpu.CompilerParams(use_tc_tiling_on_sc=False),
)(_sc_gather_scatter_body)


def _dot(a, b):
    return jnp.dot(a.astype(jnp.bfloat16), b.astype(jnp.bfloat16),
                   preferred_element_type=_f32)


def _round_bf16(a):
    return a.astype(jnp.bfloat16).astype(_f32)


def _tc_embed_body(x_ref, w_ref, b_ref, o_ref):
    o_ref[...] = (
        _dot(x_ref[...], w_ref[...]) + b_ref[...])


def _tc_layer_body(last, sp_ref, ed_ref, h_ref, we_ref, be_ref, w1_ref, b1_ref,
                   w2_ref, b2_ref, g_ref, bt_ref, o_ref):
    s_sum = sp_ref[0, :N, :] + sp_ref[1, :N, :]
    ed = ed_ref[0, :N, :] + ed_ref[1, :N, :]
    eagg = ed[:, :DE]
    deg = ed[:, DE:DE + 1]
    aggr = (s_sum + h_ref[...]
            + jnp.dot(eagg, _round_bf16(we_ref[...]), preferred_element_type=_f32,
                      precision=lax.Precision.HIGHEST)
            + (deg + 1.0) * be_ref[...])
    a1 = jnp.maximum(
        _dot(aggr, w1_ref[...]) + b1_ref[...], 0.0)
    m = _dot(a1, w2_ref[...]) + b2_ref[...]
    mean = jnp.mean(m, axis=0, keepdims=True)
    ctr = m - mean
    var = jnp.mean(ctr * ctr, axis=0, keepdims=True)
    mm = g_ref[...] * ctr * lax.rsqrt(var + 1e-5) + bt_ref[...]
    if not last:
        mm = jnp.maximum(mm, 0.0)
    o_ref[...] = mm


def _tc_embed(x, w0, b0):
    return pl.pallas_call(
        _tc_embed_body,
        out_shape=jax.ShapeDtypeStruct((N, D), _f32),
    )(x, w0, b0)


def _tc_layer(last, sp, ed, h, we, be, w1, b1, w2, b2, g, bt):
    return pl.pallas_call(
        functools.partial(_tc_layer_body, last),
        out_shape=jax.ShapeDtypeStruct((N, D), _f32),
    )(sp, ed, h, we, be, w1, b1, w2, b2, g, bt)


def kernel(x, edge_index, edge_attr, W0, b0, We0, be0, W1_0, b1_0, W2_0, b2_0,
           gamma0, beta0, We1, be1, W1_1, b1_1, W2_1, b2_1, gamma1, beta1):
    src = edge_index[0].reshape(NW, EPW)
    dst = edge_index[1].reshape(NW, EPW)
    pad = EPW_PAD - EPW
    src3 = jnp.pad(src, ((0, 0), (0, pad))).reshape(NW, NCHUNK, CH)
    dst3 = jnp.pad(dst, ((0, 0), (0, pad)), constant_values=N).reshape(
        NW, NCHUNK, CH)
    edge_attr_r = edge_attr.astype(jnp.bfloat16).astype(jnp.float32)
    ea4 = jnp.pad(edge_attr_r.reshape(NW, EPW, DE),
                  ((0, 0), (0, pad), (0, 0))).reshape(NW, NCHUNK, CH, DE)

    ed = _sc_edge_aggr(dst3, ea4)

    h = _tc_embed(x, W0, b0.reshape(1, D))

    params = [(We0, be0, W1_0, b1_0, W2_0, b2_0, gamma0, beta0),
              (We1, be1, W1_1, b1_1, W2_1, b2_1, gamma1, beta1)]
    for layer, (we, be, w1, b1, w2, b2, g, bt) in enumerate(params):
        sp = _sc_gather_scatter(h, src3, dst3)
        h = _tc_layer(layer == 1, sp, ed, h, we, be.reshape(1, D),
                      w1, b1.reshape(1, H), w2, b2.reshape(1, D),
                      g.reshape(1, D), bt.reshape(1, D))
    return h


# speedup vs baseline: 6.0787x; 1.2375x over previous
import functools

import jax
import jax.numpy as jnp
from jax import lax
from jax.experimental import pallas as pl
from jax.experimental.pallas import tpu as pltpu
from jax.experimental.pallas import tpu_sc as plsc

N = 10000
E = 320000
D = 128
DE = 16
H = 256

NC = 2
NS = 16
NW = NC * NS
EPW = E // NW
CH = 128
NCHUNK = 79
EPW_PAD = NCHUNK * CH
N_PAD = 10112
RPT = N_PAD // NS
IDXR = 32

_mesh = plsc.VectorSubcoreMesh(
    core_axis_name="c", subcore_axis_name="s", num_cores=NC, num_subcores=NS)

_f32 = jnp.float32


def _zero16():
    return jnp.zeros((16,), _f32)


def _sc_edge_aggr_body(dst3, ea2, ed_out, dstv, rows, acc, sem):
    c = lax.axis_index("c")
    s = lax.axis_index("s")
    w = c * NS + s

    one0 = jnp.where(lax.iota(jnp.int32, 16) == 0, 1.0, 0.0).astype(_f32)

    def _init_rows(i, _):
        rows[i, pl.ds(0, 16)] = _zero16()
        rows[i, pl.ds(16, 16)] = _zero16()
        return 0

    lax.fori_loop(0, CH, _init_rows, 0)

    base = s * RPT
    for off, sz in ((0, CH), (CH, CH), (2 * CH, CH), (3 * CH, CH),
                    (4 * CH, RPT - 4 * CH)):
        pltpu.sync_copy(rows.at[pl.ds(0, sz)], acc.at[pl.ds(base + off, sz)])

    def _init_ones(i, _):
        rows[i, pl.ds(16, 16)] = one0
        return 0

    lax.fori_loop(0, CH, _init_ones, 0)
    plsc.subcore_barrier()

    pltpu.sync_copy(dst3.at[w, pl.ds(0, NCHUNK)], dstv)

    def _step(k, _):
        pltpu.sync_copy(ea2.at[w, pl.ds(k * CH, CH)], rows.at[:, pl.ds(0, DE)])
        pltpu.sync_copy(rows, acc.at[dstv.at[k]], add=True)
        return 0

    lax.fori_loop(0, NCHUNK - 1, _step, 0)
    tail = EPW - (NCHUNK - 1) * CH
    def _zero_tail(i, _):
        rows[i, pl.ds(0, 16)] = _zero16()
        return 0
    lax.fori_loop(tail, CH, _zero_tail, 0)
    pltpu.sync_copy(ea2.at[w, pl.ds((NCHUNK - 1) * CH, tail)],
                    rows.at[pl.ds(0, tail), pl.ds(0, DE)])
    pltpu.sync_copy(rows, acc.at[dstv.at[NCHUNK - 1]], add=True)
    plsc.subcore_barrier()

    pltpu.sync_copy(acc.at[pl.ds(base, RPT)],
                    ed_out.at[c, pl.ds(base, RPT)])


_sc_edge_aggr = functools.partial(
    pl.kernel,
    out_type=jax.ShapeDtypeStruct((NC, N_PAD, 2 * DE), _f32),
    mesh=_mesh,
    scratch_types=[
        pltpu.VMEM((NCHUNK, CH), jnp.int32),
        pltpu.VMEM((CH, 2 * DE), _f32),
        pltpu.VMEM_SHARED((N_PAD, 2 * DE), _f32),
        pltpu.SemaphoreType.DMA,
    ],
    compiler_params=pltpu.CompilerParams(use_tc_tiling_on_sc=False),
)(_sc_edge_aggr_body)


def _sc_gather_scatter_body(h_hbm, src3, dst3, s_out,
                            srcv, dstv, rows_a, rows_b, acc, sem_a, sem_b):
    c = lax.axis_index("c")
    s = lax.axis_index("s")
    w = c * NS + s

    def _zero_rows(i, _):
        for j in range(D // 16):
            rows_a[i, pl.ds(j * 16, 16)] = _zero16()
        return 0

    lax.fori_loop(0, CH, _zero_rows, 0)

    base = s * RPT
    for off, sz in ((0, CH), (CH, CH), (2 * CH, CH), (3 * CH, CH),
                    (4 * CH, RPT - 4 * CH)):
        pltpu.sync_copy(rows_a.at[pl.ds(0, sz)], acc.at[pl.ds(base + off, sz)])

    pltpu.sync_copy(src3.at[w, pl.ds(0, IDXR)], srcv)
    pltpu.sync_copy(dst3.at[w, pl.ds(0, IDXR)], dstv)
    plsc.subcore_barrier()

    pltpu.async_copy(h_hbm.at[srcv.at[0]], rows_a, sem_a)
    pltpu.async_copy(h_hbm.at[srcv.at[1]], rows_b, sem_b)

    def _step(k, _):
        slot = lax.rem(k, IDXR)

        @pl.when(jnp.logical_and(lax.rem(k, 8) == 0, k + 16 < NCHUNK + 1))
        def _():
            off = lax.rem(k + 16, IDXR)
            pltpu.sync_copy(src3.at[w, pl.ds(k + 16, 8)], srcv.at[pl.ds(off, 8)])
            pltpu.sync_copy(dst3.at[w, pl.ds(k + 16, 8)], dstv.at[pl.ds(off, 8)])

        @pl.when(lax.rem(k, 2) == 0)
        def _():
            pltpu.make_async_copy(h_hbm.at[srcv.at[slot]], rows_a, sem_a).wait()
            pltpu.sync_copy(rows_a, acc.at[dstv.at[slot]], add=True)

            @pl.when(k + 2 < NCHUNK)
            def _():
                pltpu.async_copy(h_hbm.at[srcv.at[lax.rem(k + 2, IDXR)]],
                                 rows_a, sem_a)

        @pl.when(lax.rem(k, 2) == 1)
        def _():
            pltpu.make_async_copy(h_hbm.at[srcv.at[slot]], rows_b, sem_b).wait()
            pltpu.sync_copy(rows_b, acc.at[dstv.at[slot]], add=True)

            @pl.when(k + 2 < NCHUNK)
            def _():
                pltpu.async_copy(h_hbm.at[srcv.at[lax.rem(k + 2, IDXR)]],
                                 rows_b, sem_b)

        return 0

    lax.fori_loop(0, NCHUNK, _step, 0)
    plsc.subcore_barrier()

    for off, sz in ((0, CH), (CH, CH), (2 * CH, CH), (3 * CH, CH),
                    (4 * CH, RPT - 4 * CH)):
        pltpu.sync_copy(acc.at[pl.ds(base + off, sz)],
                        s_out.at[c, pl.ds(base + off, sz)])


_sc_gather_scatter = functools.partial(
    pl.kernel,
    out_type=jax.ShapeDtypeStruct((NC, N_PAD, D), _f32),
    mesh=_mesh,
    scratch_types=[
        pltpu.VMEM((IDXR, CH), jnp.int32),
        pltpu.VMEM((IDXR, CH), jnp.int32),
        pltpu.VMEM((CH, D), _f32),
        pltpu.VMEM((CH, D), _f32),
        pltpu.VMEM_SHARED((N_PAD, D), _f32),
        pltpu.SemaphoreType.DMA,
        pltpu.SemaphoreType.DMA,
    ],
    compiler_params=pltpu.CompilerParams(use_tc_tiling_on_sc=False),
)(_sc_gather_scatter_body)


def _dot(a, b):
    return jnp.dot(a.astype(jnp.bfloat16), b.astype(jnp.bfloat16),
                   preferred_element_type=_f32)


def _round_bf16(a):
    return a.astype(jnp.bfloat16).astype(_f32)


def _tc_embed_body(x_ref, w_ref, b_ref, o_ref):
    o_ref[...] = _dot(x_ref[...], w_ref[...]) + b_ref[...]


def _tc_layer_body(last, sp_ref, ed_ref, h_ref, we_ref, be_ref, w1_ref, b1_ref,
                   w2_ref, b2_ref, g_ref, bt_ref, o_ref):
    s_sum = sp_ref[0, :N, :] + sp_ref[1, :N, :]
    ed = ed_ref[0, :N, :] + ed_ref[1, :N, :]
    eagg = ed[:, :DE]
    deg = ed[:, DE:DE + 1]
    aggr = (s_sum + h_ref[...]
            + jnp.dot(eagg, _round_bf16(we_ref[...]), preferred_element_type=_f32,
                      precision=lax.Precision.HIGHEST)
            + (deg + 1.0) * be_ref[...])
    a1 = jnp.maximum(_dot(aggr, w1_ref[...]) + b1_ref[...], 0.0)
    m = _dot(a1, w2_ref[...]) + b2_ref[...]
    mean = jnp.mean(m, axis=0, keepdims=True)
    ctr = m - mean
    var = jnp.mean(ctr * ctr, axis=0, keepdims=True)
    mm = g_ref[...] * ctr * lax.rsqrt(var + 1e-5) + bt_ref[...]
    if not last:
        mm = jnp.maximum(mm, 0.0)
    o_ref[...] = mm


def _tc_embed(x, w0, b0):
    return pl.pallas_call(
        _tc_embed_body,
        out_shape=jax.ShapeDtypeStruct((N, D), _f32),
    )(x, w0, b0)


def _tc_layer(last, sp, ed, h, we, be, w1, b1, w2, b2, g, bt):
    return pl.pallas_call(
        functools.partial(_tc_layer_body, last),
        out_shape=jax.ShapeDtypeStruct((N, D), _f32),
    )(sp, ed, h, we, be, w1, b1, w2, b2, g, bt)


def kernel(x, edge_index, edge_attr, W0, b0, We0, be0, W1_0, b1_0, W2_0, b2_0,
           gamma0, beta0, We1, be1, W1_1, b1_1, W2_1, b2_1, gamma1, beta1):
    src = edge_index[0].reshape(NW, EPW)
    dst = edge_index[1].reshape(NW, EPW)
    pad = (NCHUNK + 1) * CH - EPW
    src3 = jnp.pad(src, ((0, 0), (0, pad))).reshape(NW, NCHUNK + 1, CH)
    dst3 = jnp.pad(dst, ((0, 0), (0, pad)),
                   constant_values=N).reshape(NW, NCHUNK + 1, CH)
    ea2 = edge_attr.astype(jnp.bfloat16).astype(jnp.float32).reshape(NW, EPW, DE)

    ed = _sc_edge_aggr(dst3, ea2)

    h = _tc_embed(x, W0, b0.reshape(1, D))

    params = [(We0, be0, W1_0, b1_0, W2_0, b2_0, gamma0, beta0),
              (We1, be1, W1_1, b1_1, W2_1, b2_1, gamma1, beta1)]
    for layer, (we, be, w1, b1, w2, b2, g, bt) in enumerate(params):
        sp = _sc_gather_scatter(h, src3, dst3)
        h = _tc_layer(layer == 1, sp, ed, h, we, be.reshape(1, D),
                      w1, b1.reshape(1, H), w2, b2.reshape(1, D),
                      g.reshape(1, D), bt.reshape(1, D))
    return h
